# baseline (speedup 1.0000x reference)
import functools

import jax
import jax.numpy as jnp
from jax import lax
from jax.experimental import pallas as pl
from jax.experimental.pallas import tpu as pltpu
from jax.experimental.pallas import tpu_sc as plsc

_T, _B, _D = 2048, 16, 1024
_T2 = _T // 2
_NC, _NS, _L = 2, 16, 16
_NW = _NC * _NS
_BLK = 16
_NBLK = _T2 // _BLK
_N = _NBLK * _B // _NW


@functools.partial(
    pl.kernel,
    out_type=jax.ShapeDtypeStruct((_T2, _B, 2 * _D), jnp.float32),
    mesh=plsc.VectorSubcoreMesh(core_axis_name="c", subcore_axis_name="s"),
    scratch_types=[
        pltpu.VMEM((_L,), jnp.int32),
        pltpu.VMEM((_B, _L), jnp.int32),
        pltpu.VMEM((2, 2, _L), jnp.int32),
        pltpu.VMEM((2, _BLK, 2 * _D), jnp.float32),
        pltpu.VMEM((_BLK, 2 * _D), jnp.float32),
        pltpu.SemaphoreType.DMA,
        pltpu.SemaphoreType.DMA,
        pltpu.SemaphoreType.DMA,
        pltpu.SemaphoreType.DMA,
    ],
)
def _pool_sc(x_hbm, lens_hbm, out_hbm,
             lens0_v, lens_v, idx_v, buf_v, zbuf_v, gsem0, gsem1, wsem0, wsem1):
    wid = lax.axis_index("s") * _NC + lax.axis_index("c")
    i_vec = lax.iota(jnp.int32, _L)
    gsems = (gsem0, gsem1)
    wsems = (wsem0, wsem1)

    pltpu.sync_copy(lens_hbm, lens0_v)
    lens_reg = lens0_v[...]
    for bb in range(_B):
        lens_v[bb] = jnp.zeros((_L,), jnp.int32) + lens_reg[bb]

    zrow = jnp.minimum(lens_reg[_B - 1], _T - 1) * _B + (_B - 1)
    idx_v[0, 0] = jnp.zeros((_L,), jnp.int32) + zrow
    pltpu.async_copy(x_hbm.at[idx_v.at[0, 0]],
                     zbuf_v.at[:, pl.ds(0, _D)], gsem0).wait()
    pltpu.async_copy(x_hbm.at[idx_v.at[0, 0]],
                     zbuf_v.at[:, pl.ds(_D, _D)], gsem0).wait()

    def item(k):
        first = wid * _BLK
        second = (_NBLK - 1 - wid) * _BLK
        half = k >> 4
        b = k & (_B - 1)
        return jnp.where(half == 0, first, second), b

    def item_valid(k):
        t2_0, b = item(k)
        newlens = lens_v[b] >> 1
        return t2_0 < newlens[0]

    def fill_idx(k, slot):
        t2_0, b = item(k)
        lens_b = lens_v[b]
        t2_vec = t2_0 + i_vec
        masked = (lens_b >> 1) <= t2_vec
        for j in range(2):
            t_nat = 2 * t2_vec + j
            t_src = jnp.where(masked, jnp.maximum(t_nat, lens_b), t_nat)
            idx_v[slot, j] = t_src * _B + b

    def gathers(slot):
        return (
            pltpu.make_async_copy(
                x_hbm.at[idx_v.at[slot, 0]],
                buf_v.at[slot, :, pl.ds(0, _D)], gsems[slot]),
            pltpu.make_async_copy(
                x_hbm.at[idx_v.at[slot, 1]],
                buf_v.at[slot, :, pl.ds(_D, _D)], gsems[slot]),
        )

    def write(k, slot, valid):
        t2_0, b = item(k)
        src = buf_v.at[slot] if valid else zbuf_v
        return pltpu.make_async_copy(
            src, out_hbm.at[pl.ds(t2_0, _BLK), b], wsems[slot])

    def start_item(k, slot):
        @pl.when(item_valid(k))
        def _():
            fill_idx(k, slot)
            for g in gathers(slot):
                g.start()

    def finish_item(k, slot):
        valid = item_valid(k)

        @pl.when(valid)
        def _():
            for g in gathers(slot):
                g.wait()
            write(k, slot, True).start()

        @pl.when(jnp.logical_not(valid))
        def _():
            write(k, slot, False).start()

    start_item(0, 0)

    @pl.loop(0, _N, step=2)
    def _pair(g):
        for p in range(2):
            k = g + p
            slot, nslot = p, 1 - p

            def prefetch():
                @pl.when(k >= 1)
                def _drain():
                    write(k - 1, nslot, True).wait()

                start_item(k + 1, nslot)

            if p == 0:
                prefetch()
            else:
                pl.when(k + 1 < _N)(prefetch)

            finish_item(k, slot)

    write(_N - 2, 0, True).wait()
    write(_N - 1, 1, True).wait()


def kernel(x, lens):
    T, B, D = x.shape
    out = _pool_sc(x.reshape(T * B, D), lens.astype(jnp.int32))
    return out, lens // 2


# speedup vs baseline: 1.3769x; 1.3769x over previous
import functools

import jax
import jax.numpy as jnp
from jax import lax
from jax.experimental import pallas as pl
from jax.experimental.pallas import tpu as pltpu
from jax.experimental.pallas import tpu_sc as plsc

_T, _B, _D = 2048, 16, 1024
_T2 = _T // 2
_NC, _NS, _L = 2, 16, 16
_NW = _NC * _NS
_BLK = 16
_NBLK = _T2 // _BLK
_N = _NBLK * _B // _NW


@functools.partial(
    pl.kernel,
    out_type=jax.ShapeDtypeStruct((_T2, _B, 2 * _D), jnp.float32),
    mesh=plsc.VectorSubcoreMesh(core_axis_name="c", subcore_axis_name="s"),
    scratch_types=[
        pltpu.VMEM((_L,), jnp.int32),
        pltpu.VMEM((_B, _L), jnp.int32),
        pltpu.VMEM((2, 2, _L), jnp.int32),
        pltpu.VMEM((2, _BLK, 2 * _D), jnp.float32),
        pltpu.VMEM((_BLK, 2 * _D), jnp.float32),
        pltpu.SemaphoreType.DMA,
        pltpu.SemaphoreType.DMA,
        pltpu.SemaphoreType.DMA,
        pltpu.SemaphoreType.DMA,
    ],
)
def _pool_sc(x_hbm, lens_hbm, zeros_hbm, out_hbm,
             lens0_v, lens_v, idx_v, buf_v, zbuf_v, gsem0, gsem1, wsem0, wsem1):
    wid = lax.axis_index("s") * _NC + lax.axis_index("c")
    i_vec = lax.iota(jnp.int32, _L)
    gsems = (gsem0, gsem1)
    wsems = (wsem0, wsem1)

    pltpu.sync_copy(lens_hbm, lens0_v)
    pltpu.sync_copy(zeros_hbm, zbuf_v)
    lens_reg = lens0_v[...]
    for bb in range(_B):
        lens_v[bb] = jnp.zeros((_L,), jnp.int32) + lens_reg[bb]

    def item(k):
        first = wid * _BLK
        second = (_NBLK - 1 - wid) * _BLK
        half = k >> 4
        b = k & (_B - 1)
        return jnp.where(half == 0, first, second), b

    def item_valid(k):
        t2_0, b = item(k)
        newlens = lens_v[b] >> 1
        return t2_0 < newlens[0]

    def fill_idx(k, slot):
        t2_0, b = item(k)
        lens_b = lens_v[b]
        t2_vec = t2_0 + i_vec
        masked = (lens_b >> 1) <= t2_vec
        for j in range(2):
            t_nat = 2 * t2_vec + j
            t_src = jnp.where(masked, jnp.maximum(t_nat, lens_b), t_nat)
            idx_v[slot, j] = t_src * _B + b

    def gathers(slot):
        return (
            pltpu.make_async_copy(
                x_hbm.at[idx_v.at[slot, 0]],
                buf_v.at[slot, :, pl.ds(0, _D)], gsems[slot]),
            pltpu.make_async_copy(
                x_hbm.at[idx_v.at[slot, 1]],
                buf_v.at[slot, :, pl.ds(_D, _D)], gsems[slot]),
        )

    def write(k, slot, valid):
        t2_0, b = item(k)
        src = buf_v.at[slot] if valid else zbuf_v
        return pltpu.make_async_copy(
            src, out_hbm.at[pl.ds(t2_0, _BLK), b], wsems[slot])

    def start_item(k, slot):
        @pl.when(item_valid(k))
        def _():
            fill_idx(k, slot)
            for g in gathers(slot):
                g.start()

    def finish_item(k, slot):
        valid = item_valid(k)

        @pl.when(valid)
        def _():
            for g in gathers(slot):
                g.wait()
            write(k, slot, True).start()

        @pl.when(jnp.logical_not(valid))
        def _():
            write(k, slot, False).start()

    start_item(0, 0)

    @pl.loop(0, _N, step=2)
    def _pair(g):
        for p in range(2):
            k = g + p
            slot, nslot = p, 1 - p

            def prefetch():
                @pl.when(k >= 1)
                def _drain():
                    write(k - 1, nslot, True).wait()

                start_item(k + 1, nslot)

            if p == 0:
                prefetch()
            else:
                pl.when(k + 1 < _N)(prefetch)

            finish_item(k, slot)

    write(_N - 2, 0, True).wait()
    write(_N - 1, 1, True).wait()


def kernel(x, lens):
    T, B, D = x.shape
    zeros = jnp.zeros((_BLK, 2 * D), jnp.float32)
    out = _pool_sc(x.reshape(T * B, D), lens.astype(jnp.int32), zeros)
    return out, lens // 2
